# baseline (speedup 1.0000x reference)
import functools

import jax
import jax.numpy as jnp
from jax import lax
from jax.experimental import pallas as pl
from jax.experimental.pallas import tpu as pltpu
from jax.experimental.pallas import tpu_sc as plsc

_BATCH = 4096
_SEQ = 200
_EMBED = 64
_CLASSES = 50
_LANES = 16

_NC = 2
_NS = 16
_NW = _NC * _NS
_RPW = _BATCH // _NW
_G0 = 128
_G1 = _SEQ - _G0


def _pool_body(x_hbm, tab_hbm, out_hbm, idx_v, rows_v, pooled_v, sem0, sem1):
    wid = lax.axis_index("s") * _NC + lax.axis_index("c")
    base = pl.multiple_of(wid * _RPW, _RPW)

    pltpu.sync_copy(x_hbm.at[pl.ds(base * _SEQ, _RPW * _SEQ)], idx_v)

    sems = (sem0, sem1)

    def issue(r, slot):
        off = pl.multiple_of(r * _SEQ, 8)
        pltpu.async_copy(tab_hbm.at[idx_v.at[pl.ds(off, _G0)]],
                         rows_v.at[slot, pl.ds(0, _G0)], sems[slot])
        pltpu.async_copy(tab_hbm.at[idx_v.at[pl.ds(off + _G0, _G1)]],
                         rows_v.at[slot, pl.ds(_G0, _G1)], sems[slot])

    def wait(slot):
        pltpu.make_async_copy(tab_hbm.at[idx_v.at[pl.ds(0, _G0)]],
                              rows_v.at[slot, pl.ds(0, _G0)], sems[slot]).wait()
        pltpu.make_async_copy(tab_hbm.at[idx_v.at[pl.ds(0, _G1)]],
                              rows_v.at[slot, pl.ds(_G0, _G1)], sems[slot]).wait()

    issue(0, 0)

    inv = jnp.float32(1.0 / _SEQ)

    def step(r, slot, nslot):
        @pl.when(r + 1 < _RPW)
        def _():
            issue(r + 1, nslot)

        wait(slot)

        zero = jnp.zeros((_LANES,), jnp.float32)

        def body(j, accs):
            return tuple(accs[d] + rows_v[slot, j, pl.ds(d * _LANES, _LANES)]
                         for d in range(_EMBED // _LANES))

        accs = lax.fori_loop(0, _SEQ, body, (zero,) * (_EMBED // _LANES))
        for d in range(_EMBED // _LANES):
            pooled_v[r, pl.ds(d * _LANES, _LANES)] = accs[d] * inv

    def outer(i, carry):
        step(2 * i, 0, 1)
        step(2 * i + 1, 1, 0)
        return carry

    lax.fori_loop(0, _RPW // 2, outer, 0)

    pltpu.sync_copy(pooled_v, out_hbm.at[pl.ds(base, _RPW)])


_pool = functools.partial(
    pl.kernel,
    mesh=plsc.VectorSubcoreMesh(core_axis_name="c", subcore_axis_name="s"),
    compiler_params=pltpu.CompilerParams(use_tc_tiling_on_sc=False),
    out_type=jax.ShapeDtypeStruct((_BATCH, _EMBED), jnp.float32),
    scratch_types=[
        pltpu.VMEM((_RPW * _SEQ,), jnp.int32),
        pltpu.VMEM((2, _SEQ, _EMBED), jnp.float32),
        pltpu.VMEM((_RPW, _EMBED), jnp.float32),
        pltpu.SemaphoreType.DMA,
        pltpu.SemaphoreType.DMA,
    ],
)(_pool_body)


def _fc_body(p_ref, w_ref, b_ref, o_ref):
    o_ref[...] = (
        jnp.dot(p_ref[...], w_ref[...], preferred_element_type=jnp.float32)
        + b_ref[...]
    )


def kernel(x, emb_table, fc_w, fc_b):
    pooled = _pool(x.reshape(-1), emb_table)
    return pl.pallas_call(
        _fc_body,
        out_shape=jax.ShapeDtypeStruct((_BATCH, _CLASSES), jnp.float32),
    )(pooled, fc_w, fc_b.reshape(1, _CLASSES))
